# baseline (speedup 1.0000x reference)
import functools

import jax
import jax.numpy as jnp
from jax import lax
from jax.experimental import pallas as pl
from jax.experimental.pallas import tpu as pltpu
from jax.experimental.pallas import tpu_sc as plsc

B, C, S, D_MODEL = 4, 8, 128, 768
D_FF = 3072
E = 8
CS = C * S
D3 = 3 * D_MODEL
BF = 1024
J = D_FF // BF
LANES = 16
CHUNKS = D3 // LANES


def _means_body(x_ref, bl_ref, spa_rW_ref, spe_rW_ref,
                feats_ref, spa_rWr_ref, spe_rWr_ref):
    inv = jnp.float32(1.0 / CS)
    xm = jnp.sum(x_ref[...].reshape(B, CS, D_MODEL), axis=1) * inv
    bm = jnp.sum(bl_ref[...].reshape(B, CS, D_MODEL), axis=1) * inv
    feats = jnp.concatenate([bm, xm, xm - bm], axis=-1)
    rnd = lambda v: v.astype(jnp.bfloat16).astype(jnp.float32)
    feats_ref[...] = rnd(feats)
    spa_rWr_ref[...] = rnd(spa_rW_ref[...])
    spe_rWr_ref[...] = rnd(spe_rW_ref[...])


def _sc_router_body(feats_hbm, spa_rW_hbm, spe_rW_hbm, spa_rb_hbm, spe_rb_hbm,
                    idx_a_hbm, gate_a_hbm, idx_b_hbm, gate_b_hbm,
                    feats_v, rw_v, rb_v, oi_v, og_v):
    info = plsc.get_sparse_core_info()
    nc = info.num_cores
    wid = lax.axis_index("s") * nc + lax.axis_index("c")
    lane = lax.iota(jnp.int32, 16)

    dnums = lax.GatherDimensionNumbers(
        offset_dims=(), collapsed_slice_dims=(0,), start_index_map=(0,))

    def tree_reduce(v, op):
        for off in (8, 4, 2, 1):
            idxv = ((lane + off) & 15).reshape(16, 1)
            rot = lax.gather(v, idxv, dnums, slice_sizes=(1,),
                             mode=lax.GatherScatterMode.PROMISE_IN_BOUNDS)
            v = op(v, rot)
        return v

    def route_one(b, rw_hbm, rb_hbm, idx_hbm, gate_hbm):
        pltpu.sync_copy(feats_hbm.at[b], feats_v)
        pltpu.sync_copy(rw_hbm, rw_v)
        pltpu.sync_copy(rb_hbm, rb_v)
        logits = jnp.zeros((16,), jnp.float32)
        for e in range(E):
            def body(k, acc):
                fa = feats_v[pl.ds(k * LANES, LANES)]
                wa = rw_v[e, pl.ds(k * LANES, LANES)]
                return acc + fa * wa
            acc = lax.fori_loop(0, CHUNKS, body, jnp.zeros((16,), jnp.float32))
            se = tree_reduce(acc, jnp.add)
            logits = jnp.where(lane == e, se, logits)
        logits = logits + rb_v[...]
        lm = jnp.where(lane < E, logits, jnp.float32(-1e30))
        m = tree_reduce(lm, jnp.maximum)
        p = jnp.exp(lm - m)
        gate = jnp.float32(1.0) / tree_reduce(p, jnp.add)
        is_max = lm == m
        idx = tree_reduce(jnp.where(is_max, lane, jnp.int32(16)), jnp.minimum)
        oi_v[...] = idx
        og_v[...] = gate
        pltpu.sync_copy(oi_v, idx_hbm.at[b])
        pltpu.sync_copy(og_v, gate_hbm.at[b])

    @pl.when(wid < B)
    def _spa():
        route_one(wid, spa_rW_hbm, spa_rb_hbm, idx_a_hbm, gate_a_hbm)

    @pl.when((wid >= B) & (wid < 2 * B))
    def _spe():
        route_one(wid - B, spe_rW_hbm, spe_rb_hbm, idx_b_hbm, gate_b_hbm)


def _ffn_body(idx_a_ref, idx_b_ref, gate_a_ref, gate_b_ref,
              x_ref, w1s_ref, b1s_ref, w2s_ref,
              w1a_ref, w2a_ref, w1b_ref, w2b_ref,
              b1a_ref, b1b_ref, b2s_ref, b2a_ref, b2b_ref, o_ref):
    b = pl.program_id(0)
    j = pl.program_id(1)
    x = x_ref[0]
    ga = gate_a_ref[b]
    gb = gate_b_ref[b]
    cdims = (((1,), (1,)), ((), ()))

    def mm(a, w):
        return lax.dot_general(a, w, cdims, preferred_element_type=jnp.float32)

    h_s = jax.nn.gelu(mm(x, w1s_ref[...]) + b1s_ref[0, 0, :])
    h_a = jax.nn.gelu(mm(x, w1a_ref[0]) + b1a_ref[0, 0, :]) * ga
    h_b = jax.nn.gelu(mm(x, w1b_ref[0]) + b1b_ref[0, 0, :]) * gb

    acc = mm(h_s, w2s_ref[...]) + mm(h_a, w2a_ref[0]) + mm(h_b, w2b_ref[0])

    @pl.when(j == 0)
    def _init():
        b2 = (b2s_ref[0, 0, :] + ga * b2a_ref[0, 0, :]
              + gb * b2b_ref[0, 0, :])
        o_ref[0] = acc + b2

    @pl.when(j > 0)
    def _acc():
        o_ref[0] += acc


@jax.jit
def kernel(x, baseline, shared_W1, shared_b1, shared_W2, shared_b2,
           spa_rW, spa_rb, spa_W1, spa_b1, spa_W2, spa_b2,
           spe_rW, spe_rb, spe_W1, spe_b1, spe_W2, spe_b2):
    f32 = jnp.float32
    x3 = x.reshape(B, CS, D_MODEL)
    bl3 = baseline.reshape(B, CS, D_MODEL)

    feats, spa_rWr, spe_rWr = pl.pallas_call(
        _means_body,
        out_shape=(
            jax.ShapeDtypeStruct((B, D3), f32),
            jax.ShapeDtypeStruct((E, D3), f32),
            jax.ShapeDtypeStruct((E, D3), f32),
        ),
    )(x3, bl3, spa_rW, spe_rW)

    rb_a16 = jnp.pad(spa_rb, (0, 16 - E))
    rb_b16 = jnp.pad(spe_rb, (0, 16 - E))

    mesh = plsc.VectorSubcoreMesh(core_axis_name="c", subcore_axis_name="s")
    sc_router = functools.partial(
        pl.kernel, mesh=mesh,
        out_type=(
            jax.ShapeDtypeStruct((B, 16), jnp.int32),
            jax.ShapeDtypeStruct((B, 16), f32),
            jax.ShapeDtypeStruct((B, 16), jnp.int32),
            jax.ShapeDtypeStruct((B, 16), f32),
        ),
        scratch_types=[
            pltpu.VMEM((D3,), f32),
            pltpu.VMEM((E, D3), f32),
            pltpu.VMEM((16,), f32),
            pltpu.VMEM((16,), jnp.int32),
            pltpu.VMEM((16,), f32),
        ],
    )(_sc_router_body)
    idx_a_g, gate_a_g, idx_b_g, gate_b_g = sc_router(
        feats, spa_rWr, spe_rWr, rb_a16, rb_b16)

    idx_a = idx_a_g[:, 0]
    idx_b = idx_b_g[:, 0]
    gate_a = gate_a_g[:, 0]
    gate_b = gate_b_g[:, 0]

    grid_spec = pltpu.PrefetchScalarGridSpec(
        num_scalar_prefetch=4,
        grid=(B, J),
        in_specs=[
            pl.BlockSpec((1, CS, D_MODEL), lambda b, j, ia, ib, ga, gb: (b, 0, 0)),
            pl.BlockSpec((BF, D_MODEL), lambda b, j, ia, ib, ga, gb: (j, 0)),
            pl.BlockSpec((1, 1, BF), lambda b, j, ia, ib, ga, gb: (0, 0, j)),
            pl.BlockSpec((D_MODEL, BF), lambda b, j, ia, ib, ga, gb: (0, j)),
            pl.BlockSpec((1, BF, D_MODEL),
                         lambda b, j, ia, ib, ga, gb: (ia[b], j, 0)),
            pl.BlockSpec((1, D_MODEL, BF),
                         lambda b, j, ia, ib, ga, gb: (ia[b], 0, j)),
            pl.BlockSpec((1, BF, D_MODEL),
                         lambda b, j, ia, ib, ga, gb: (ib[b], j, 0)),
            pl.BlockSpec((1, D_MODEL, BF),
                         lambda b, j, ia, ib, ga, gb: (ib[b], 0, j)),
            pl.BlockSpec((1, 1, BF), lambda b, j, ia, ib, ga, gb: (ia[b], 0, j)),
            pl.BlockSpec((1, 1, BF), lambda b, j, ia, ib, ga, gb: (ib[b], 0, j)),
            pl.BlockSpec((1, 1, D_MODEL), lambda b, j, ia, ib, ga, gb: (0, 0, 0)),
            pl.BlockSpec((1, 1, D_MODEL),
                         lambda b, j, ia, ib, ga, gb: (ia[b], 0, 0)),
            pl.BlockSpec((1, 1, D_MODEL),
                         lambda b, j, ia, ib, ga, gb: (ib[b], 0, 0)),
        ],
        out_specs=pl.BlockSpec((1, CS, D_MODEL),
                               lambda b, j, ia, ib, ga, gb: (b, 0, 0)),
    )

    out = pl.pallas_call(
        _ffn_body,
        grid_spec=grid_spec,
        out_shape=jax.ShapeDtypeStruct((B, CS, D_MODEL), f32),
        compiler_params=pltpu.CompilerParams(
            dimension_semantics=("arbitrary", "arbitrary"),
            vmem_limit_bytes=100 * 1024 * 1024),
    )(idx_a, idx_b, gate_a, gate_b,
      x3, shared_W1, shared_b1.reshape(1, 1, D_FF), shared_W2,
      spa_W1, spa_W2, spe_W1, spe_W2,
      spa_b1.reshape(E, 1, D_FF), spe_b1.reshape(E, 1, D_FF),
      shared_b2.reshape(1, 1, D_MODEL),
      spa_b2.reshape(E, 1, D_MODEL), spe_b2.reshape(E, 1, D_MODEL))

    return out.reshape(B, C, S, D_MODEL)


# speedup vs baseline: 1.0395x; 1.0395x over previous
import functools

import jax
import jax.numpy as jnp
from jax import lax
from jax.experimental import pallas as pl
from jax.experimental.pallas import tpu as pltpu
from jax.experimental.pallas import tpu_sc as plsc

B, C, S, D_MODEL = 4, 8, 128, 768
D_FF = 3072
E = 8
CS = C * S
D3 = 3 * D_MODEL
BF = 1024
J = D_FF // BF
LANES = 16
CHUNKS = D3 // LANES


def _means_body(x_ref, bl_ref, spa_rW_ref, spa_rb_ref, spe_rW_ref, spe_rb_ref,
                logits_ref):
    inv = jnp.float32(1.0 / CS)
    xm = jnp.sum(x_ref[...].reshape(B, CS, D_MODEL), axis=1) * inv
    bm = jnp.sum(bl_ref[...].reshape(B, CS, D_MODEL), axis=1) * inv
    feats = jnp.concatenate([bm, xm, xm - bm], axis=-1)
    cdims = (((1,), (1,)), ((), ()))
    la = lax.dot_general(feats, spa_rW_ref[...], cdims,
                         preferred_element_type=jnp.float32) + spa_rb_ref[0]
    lb = lax.dot_general(feats, spe_rW_ref[...], cdims,
                         preferred_element_type=jnp.float32) + spe_rb_ref[0]
    pad = jnp.full((B, 16 - E), -1e30, jnp.float32)
    logits_ref[...] = jnp.concatenate(
        [jnp.concatenate([la, pad], axis=1),
         jnp.concatenate([lb, pad], axis=1)], axis=0)


def _sc_router_body(logits_hbm, idx_hbm, gate_hbm, lv, oi_v, og_v):
    info = plsc.get_sparse_core_info()
    nc = info.num_cores
    wid = lax.axis_index("s") * nc + lax.axis_index("c")
    lane = lax.iota(jnp.int32, 16)

    dnums = lax.GatherDimensionNumbers(
        offset_dims=(), collapsed_slice_dims=(0,), start_index_map=(0,))

    def tree_reduce(v, op):
        for off in (8, 4, 2, 1):
            idxv = ((lane + off) & 15).reshape(16, 1)
            rot = lax.gather(v, idxv, dnums, slice_sizes=(1,),
                             mode=lax.GatherScatterMode.PROMISE_IN_BOUNDS)
            v = op(v, rot)
        return v

    @pl.when(wid < 2 * B)
    def _route():
        pltpu.sync_copy(logits_hbm.at[wid], lv)
        lm = lv[...]
        m = tree_reduce(lm, jnp.maximum)
        p = jnp.exp(lm - m)
        gate = jnp.float32(1.0) / tree_reduce(p, jnp.add)
        is_max = lm == m
        idx = tree_reduce(jnp.where(is_max, lane, jnp.int32(16)), jnp.minimum)
        oi_v[...] = idx
        og_v[...] = gate
        pltpu.sync_copy(oi_v, idx_hbm.at[wid])
        pltpu.sync_copy(og_v, gate_hbm.at[wid])


def _ffn_body(idx_a_ref, idx_b_ref, gate_a_ref, gate_b_ref,
              x_ref, w1s_ref, b1s_ref, w2s_ref,
              w1a_ref, w2a_ref, w1b_ref, w2b_ref,
              b1a_ref, b1b_ref, b2s_ref, b2a_ref, b2b_ref, o_ref):
    b = pl.program_id(0)
    j = pl.program_id(1)
    x = x_ref[0]
    ga = gate_a_ref[b]
    gb = gate_b_ref[b]
    cdims = (((1,), (1,)), ((), ()))

    def mm(a, w):
        return lax.dot_general(a, w, cdims, preferred_element_type=jnp.float32)

    h_s = jax.nn.gelu(mm(x, w1s_ref[...]) + b1s_ref[0, 0, :])
    h_a = jax.nn.gelu(mm(x, w1a_ref[0]) + b1a_ref[0, 0, :]) * ga
    h_b = jax.nn.gelu(mm(x, w1b_ref[0]) + b1b_ref[0, 0, :]) * gb

    acc = mm(h_s, w2s_ref[...]) + mm(h_a, w2a_ref[0]) + mm(h_b, w2b_ref[0])

    @pl.when(j == 0)
    def _init():
        b2 = (b2s_ref[0, 0, :] + ga * b2a_ref[0, 0, :]
              + gb * b2b_ref[0, 0, :])
        o_ref[0] = acc + b2

    @pl.when(j > 0)
    def _acc():
        o_ref[0] += acc


@jax.jit
def kernel(x, baseline, shared_W1, shared_b1, shared_W2, shared_b2,
           spa_rW, spa_rb, spa_W1, spa_b1, spa_W2, spa_b2,
           spe_rW, spe_rb, spe_W1, spe_b1, spe_W2, spe_b2):
    f32 = jnp.float32
    x3 = x.reshape(B, CS, D_MODEL)
    bl3 = baseline.reshape(B, CS, D_MODEL)

    logits = pl.pallas_call(
        _means_body,
        out_shape=jax.ShapeDtypeStruct((2 * B, 16), f32),
    )(x3, bl3, spa_rW, spa_rb.reshape(1, E), spe_rW, spe_rb.reshape(1, E))

    mesh = plsc.VectorSubcoreMesh(core_axis_name="c", subcore_axis_name="s")
    sc_router = functools.partial(
        pl.kernel, mesh=mesh,
        out_type=(
            jax.ShapeDtypeStruct((2 * B, 16), jnp.int32),
            jax.ShapeDtypeStruct((2 * B, 16), f32),
        ),
        scratch_types=[
            pltpu.VMEM((16,), f32),
            pltpu.VMEM((16,), jnp.int32),
            pltpu.VMEM((16,), f32),
        ],
    )(_sc_router_body)
    idx_g, gate_g = sc_router(logits)

    idx_a = idx_g[:B, 0]
    idx_b = idx_g[B:, 0]
    gate_a = gate_g[:B, 0]
    gate_b = gate_g[B:, 0]

    grid_spec = pltpu.PrefetchScalarGridSpec(
        num_scalar_prefetch=4,
        grid=(B, J),
        in_specs=[
            pl.BlockSpec((1, CS, D_MODEL), lambda b, j, ia, ib, ga, gb: (b, 0, 0)),
            pl.BlockSpec((BF, D_MODEL), lambda b, j, ia, ib, ga, gb: (j, 0)),
            pl.BlockSpec((1, 1, BF), lambda b, j, ia, ib, ga, gb: (0, 0, j)),
            pl.BlockSpec((D_MODEL, BF), lambda b, j, ia, ib, ga, gb: (0, j)),
            pl.BlockSpec((1, BF, D_MODEL),
                         lambda b, j, ia, ib, ga, gb: (ia[b], j, 0)),
            pl.BlockSpec((1, D_MODEL, BF),
                         lambda b, j, ia, ib, ga, gb: (ia[b], 0, j)),
            pl.BlockSpec((1, BF, D_MODEL),
                         lambda b, j, ia, ib, ga, gb: (ib[b], j, 0)),
            pl.BlockSpec((1, D_MODEL, BF),
                         lambda b, j, ia, ib, ga, gb: (ib[b], 0, j)),
            pl.BlockSpec((1, 1, BF), lambda b, j, ia, ib, ga, gb: (ia[b], 0, j)),
            pl.BlockSpec((1, 1, BF), lambda b, j, ia, ib, ga, gb: (ib[b], 0, j)),
            pl.BlockSpec((1, 1, D_MODEL), lambda b, j, ia, ib, ga, gb: (0, 0, 0)),
            pl.BlockSpec((1, 1, D_MODEL),
                         lambda b, j, ia, ib, ga, gb: (ia[b], 0, 0)),
            pl.BlockSpec((1, 1, D_MODEL),
                         lambda b, j, ia, ib, ga, gb: (ib[b], 0, 0)),
        ],
        out_specs=pl.BlockSpec((1, CS, D_MODEL),
                               lambda b, j, ia, ib, ga, gb: (b, 0, 0)),
    )

    out = pl.pallas_call(
        _ffn_body,
        grid_spec=grid_spec,
        out_shape=jax.ShapeDtypeStruct((B, CS, D_MODEL), f32),
        compiler_params=pltpu.CompilerParams(
            dimension_semantics=("arbitrary", "arbitrary"),
            vmem_limit_bytes=100 * 1024 * 1024),
    )(idx_a, idx_b, gate_a, gate_b,
      x3, shared_W1, shared_b1.reshape(1, 1, D_FF), shared_W2,
      spa_W1, spa_W2, spe_W1, spe_W2,
      spa_b1.reshape(E, 1, D_FF), spe_b1.reshape(E, 1, D_FF),
      shared_b2.reshape(1, 1, D_MODEL),
      spa_b2.reshape(E, 1, D_MODEL), spe_b2.reshape(E, 1, D_MODEL))

    return out.reshape(B, C, S, D_MODEL)


# speedup vs baseline: 1.0535x; 1.0134x over previous
import functools

import jax
import jax.numpy as jnp
from jax import lax
from jax.experimental import pallas as pl
from jax.experimental.pallas import tpu as pltpu
from jax.experimental.pallas import tpu_sc as plsc

B, C, S, D_MODEL = 4, 8, 128, 768
D_FF = 3072
E = 8
CS = C * S
D3 = 3 * D_MODEL
BF = 1024
J = D_FF // BF
LANES = 16
CHUNKS = D3 // LANES


def _means_body(x_ref, bl_ref, spa_rW_ref, spa_rb_ref, spe_rW_ref, spe_rb_ref,
                logits_ref):
    inv = jnp.float32(1.0 / CS)
    xm = jnp.sum(x_ref[...].reshape(B, CS, D_MODEL), axis=1) * inv
    bm = jnp.sum(bl_ref[...].reshape(B, CS, D_MODEL), axis=1) * inv
    feats = jnp.concatenate([bm, xm, xm - bm], axis=-1)
    cdims = (((1,), (1,)), ((), ()))
    la = lax.dot_general(feats, spa_rW_ref[...], cdims,
                         preferred_element_type=jnp.float32) + spa_rb_ref[0]
    lb = lax.dot_general(feats, spe_rW_ref[...], cdims,
                         preferred_element_type=jnp.float32) + spe_rb_ref[0]
    pad = jnp.full((B, 16 - E), -1e30, jnp.float32)
    logits_ref[...] = jnp.concatenate(
        [jnp.concatenate([la, pad], axis=1),
         jnp.concatenate([lb, pad], axis=1)], axis=0)


def _sc_router_body(logits_hbm, idx_hbm, gate_hbm, lv, oi_v, og_v):
    info = plsc.get_sparse_core_info()
    nc = info.num_cores
    wid = lax.axis_index("s") * nc + lax.axis_index("c")
    lane = lax.iota(jnp.int32, 16)

    dnums = lax.GatherDimensionNumbers(
        offset_dims=(), collapsed_slice_dims=(0,), start_index_map=(0,))

    def tree_reduce(v, op):
        for off in (8, 4, 2, 1):
            idxv = ((lane + off) & 15).reshape(16, 1)
            rot = lax.gather(v, idxv, dnums, slice_sizes=(1,),
                             mode=lax.GatherScatterMode.PROMISE_IN_BOUNDS)
            v = op(v, rot)
        return v

    @pl.when(wid < 2 * B)
    def _route():
        pltpu.sync_copy(logits_hbm.at[wid], lv)
        lm = lv[...]
        m = tree_reduce(lm, jnp.maximum)
        p = jnp.exp(lm - m)
        gate = jnp.float32(1.0) / tree_reduce(p, jnp.add)
        is_max = lm == m
        idx = tree_reduce(jnp.where(is_max, lane, jnp.int32(16)), jnp.minimum)
        oi_v[...] = idx
        og_v[...] = gate
        pltpu.sync_copy(oi_v, idx_hbm.at[wid])
        pltpu.sync_copy(og_v, gate_hbm.at[wid])


def _ffn_body(idx_ref, gate_ref,
              x_ref, w1s_ref, b1s_ref, w2s_ref,
              w1a_ref, w2a_ref, w1b_ref, w2b_ref,
              b1a_ref, b1b_ref, b2s_ref, b2a_ref, b2b_ref, o_ref):
    b = pl.program_id(0)
    j = pl.program_id(1)
    x = x_ref[0]
    ga = gate_ref[b, 0]
    gb = gate_ref[B + b, 0]
    cdims = (((1,), (1,)), ((), ()))

    def mm(a, w):
        return lax.dot_general(a, w, cdims, preferred_element_type=jnp.float32)

    h_s = jax.nn.gelu(mm(x, w1s_ref[...]) + b1s_ref[0, 0, :])
    h_a = jax.nn.gelu(mm(x, w1a_ref[0]) + b1a_ref[0, 0, :]) * ga
    h_b = jax.nn.gelu(mm(x, w1b_ref[0]) + b1b_ref[0, 0, :]) * gb

    acc = mm(h_s, w2s_ref[...]) + mm(h_a, w2a_ref[0]) + mm(h_b, w2b_ref[0])

    @pl.when(j == 0)
    def _init():
        b2 = (b2s_ref[0, 0, :] + ga * b2a_ref[0, 0, :]
              + gb * b2b_ref[0, 0, :])
        o_ref[0] = acc + b2

    @pl.when(j > 0)
    def _acc():
        o_ref[0] += acc


@jax.jit
def kernel(x, baseline, shared_W1, shared_b1, shared_W2, shared_b2,
           spa_rW, spa_rb, spa_W1, spa_b1, spa_W2, spa_b2,
           spe_rW, spe_rb, spe_W1, spe_b1, spe_W2, spe_b2):
    f32 = jnp.float32
    x3 = x.reshape(B, CS, D_MODEL)
    bl3 = baseline.reshape(B, CS, D_MODEL)

    logits = pl.pallas_call(
        _means_body,
        out_shape=jax.ShapeDtypeStruct((2 * B, 16), f32),
    )(x3, bl3, spa_rW, spa_rb.reshape(1, E), spe_rW, spe_rb.reshape(1, E))

    mesh = plsc.VectorSubcoreMesh(core_axis_name="c", subcore_axis_name="s")
    sc_router = functools.partial(
        pl.kernel, mesh=mesh,
        out_type=(
            jax.ShapeDtypeStruct((2 * B, 16), jnp.int32),
            jax.ShapeDtypeStruct((2 * B, 16), f32),
        ),
        scratch_types=[
            pltpu.VMEM((16,), f32),
            pltpu.VMEM((16,), jnp.int32),
            pltpu.VMEM((16,), f32),
        ],
    )(_sc_router_body)
    idx_g, gate_g = sc_router(logits)

    grid_spec = pltpu.PrefetchScalarGridSpec(
        num_scalar_prefetch=2,
        grid=(B, J),
        in_specs=[
            pl.BlockSpec((1, CS, D_MODEL), lambda b, j, ig, gg: (b, 0, 0)),
            pl.BlockSpec((BF, D_MODEL), lambda b, j, ig, gg: (j, 0)),
            pl.BlockSpec((1, 1, BF), lambda b, j, ig, gg: (0, 0, j)),
            pl.BlockSpec((D_MODEL, BF), lambda b, j, ig, gg: (0, j)),
            pl.BlockSpec((1, BF, D_MODEL),
                         lambda b, j, ig, gg: (ig[b, 0], j, 0)),
            pl.BlockSpec((1, D_MODEL, BF),
                         lambda b, j, ig, gg: (ig[b, 0], 0, j)),
            pl.BlockSpec((1, BF, D_MODEL),
                         lambda b, j, ig, gg: (ig[B + b, 0], j, 0)),
            pl.BlockSpec((1, D_MODEL, BF),
                         lambda b, j, ig, gg: (ig[B + b, 0], 0, j)),
            pl.BlockSpec((1, 1, BF), lambda b, j, ig, gg: (ig[b, 0], 0, j)),
            pl.BlockSpec((1, 1, BF), lambda b, j, ig, gg: (ig[B + b, 0], 0, j)),
            pl.BlockSpec((1, 1, D_MODEL), lambda b, j, ig, gg: (0, 0, 0)),
            pl.BlockSpec((1, 1, D_MODEL),
                         lambda b, j, ig, gg: (ig[b, 0], 0, 0)),
            pl.BlockSpec((1, 1, D_MODEL),
                         lambda b, j, ig, gg: (ig[B + b, 0], 0, 0)),
        ],
        out_specs=pl.BlockSpec((1, CS, D_MODEL),
                               lambda b, j, ig, gg: (b, 0, 0)),
    )

    out = pl.pallas_call(
        _ffn_body,
        grid_spec=grid_spec,
        out_shape=jax.ShapeDtypeStruct((B, CS, D_MODEL), f32),
        compiler_params=pltpu.CompilerParams(
            dimension_semantics=("arbitrary", "arbitrary"),
            vmem_limit_bytes=100 * 1024 * 1024),
    )(idx_g, gate_g,
      x3, shared_W1, shared_b1.reshape(1, 1, D_FF), shared_W2,
      spa_W1, spa_W2, spe_W1, spe_W2,
      spa_b1.reshape(E, 1, D_FF), spe_b1.reshape(E, 1, D_FF),
      shared_b2.reshape(1, 1, D_MODEL),
      spa_b2.reshape(E, 1, D_MODEL), spe_b2.reshape(E, 1, D_MODEL))

    return out.reshape(B, C, S, D_MODEL)


# speedup vs baseline: 1.0566x; 1.0029x over previous
import functools

import jax
import jax.numpy as jnp
from jax import lax
from jax.experimental import pallas as pl
from jax.experimental.pallas import tpu as pltpu
from jax.experimental.pallas import tpu_sc as plsc

B, C, S, D_MODEL = 4, 8, 128, 768
D_FF = 3072
E = 8
CS = C * S
BF = 1024
J = D_FF // BF


def _means_body(x_ref, bl_ref, spa_rW_ref, spa_rb_ref, spe_rW_ref, spe_rb_ref,
                logits_ref):
    inv = jnp.float32(1.0 / CS)
    xm = jnp.sum(x_ref[...].reshape(B, CS, D_MODEL), axis=1) * inv
    bm = jnp.sum(bl_ref[...].reshape(B, CS, D_MODEL), axis=1) * inv
    feats = jnp.concatenate([bm, xm, xm - bm], axis=-1)
    cdims = (((1,), (1,)), ((), ()))
    la = lax.dot_general(feats, spa_rW_ref[...], cdims,
                         preferred_element_type=jnp.float32) + spa_rb_ref[0]
    lb = lax.dot_general(feats, spe_rW_ref[...], cdims,
                         preferred_element_type=jnp.float32) + spe_rb_ref[0]
    pad = jnp.full((B, 16 - E), -1e30, jnp.float32)
    logits_ref[...] = jnp.concatenate(
        [jnp.concatenate([la, pad], axis=1),
         jnp.concatenate([lb, pad], axis=1)], axis=0)


def _sc_router_body(logits_hbm, idx_hbm, gate_hbm, lv, oi_v, og_v):
    info = plsc.get_sparse_core_info()
    nc = info.num_cores
    wid = lax.axis_index("s") * nc + lax.axis_index("c")
    lane = lax.iota(jnp.int32, 16)

    dnums = lax.GatherDimensionNumbers(
        offset_dims=(), collapsed_slice_dims=(0,), start_index_map=(0,))

    def tree_reduce(v, op):
        for off in (8, 4, 2, 1):
            idxv = ((lane + off) & 15).reshape(16, 1)
            rot = lax.gather(v, idxv, dnums, slice_sizes=(1,),
                             mode=lax.GatherScatterMode.PROMISE_IN_BOUNDS)
            v = op(v, rot)
        return v

    @pl.when(wid < 2 * B)
    def _route():
        pltpu.sync_copy(logits_hbm.at[wid], lv)
        lm = lv[...]
        m = tree_reduce(lm, jnp.maximum)
        p = jnp.exp(lm - m)
        gate = jnp.float32(1.0) / tree_reduce(p, jnp.add)
        is_max = lm == m
        idx = tree_reduce(jnp.where(is_max, lane, jnp.int32(16)), jnp.minimum)
        oi_v[...] = idx
        og_v[...] = gate
        pltpu.sync_copy(oi_v, idx_hbm.at[wid])
        pltpu.sync_copy(og_v, gate_hbm.at[wid])


def _ffn_body(idx_ref, gate_ref,
              x_ref, w1s_ref, b1s_ref, w2s_ref,
              w1a_ref, w2a_ref, w1b_ref, w2b_ref,
              b1a_ref, b1b_ref, b2s_ref, b2a_ref, b2b_ref, o_ref):
    b = pl.program_id(0)
    j = pl.program_id(1)
    x = x_ref[0]
    ga = gate_ref[b, 0]
    gb = gate_ref[B + b, 0]
    cdims = (((1,), (1,)), ((), ()))

    def mm(a, w):
        return lax.dot_general(a, w, cdims, preferred_element_type=jnp.float32)

    h_s = jax.nn.gelu(mm(x, w1s_ref[...]) + b1s_ref[0, 0, :])
    h_a = jax.nn.gelu(mm(x, w1a_ref[0]) + b1a_ref[0, 0, :]) * ga
    h_b = jax.nn.gelu(mm(x, w1b_ref[0]) + b1b_ref[0, 0, :]) * gb

    acc = mm(h_s, w2s_ref[...]) + mm(h_a, w2a_ref[0]) + mm(h_b, w2b_ref[0])

    @pl.when(j == 0)
    def _init():
        b2 = (b2s_ref[0, 0, :] + ga * b2a_ref[0, 0, :]
              + gb * b2b_ref[0, 0, :])
        o_ref[0] = acc + b2

    @pl.when(j > 0)
    def _acc():
        o_ref[0] += acc


@jax.jit
def kernel(x, baseline, shared_W1, shared_b1, shared_W2, shared_b2,
           spa_rW, spa_rb, spa_W1, spa_b1, spa_W2, spa_b2,
           spe_rW, spe_rb, spe_W1, spe_b1, spe_W2, spe_b2):
    f32 = jnp.float32
    x3 = x.reshape(B, CS, D_MODEL)
    bl3 = baseline.reshape(B, CS, D_MODEL)

    logits = pl.pallas_call(
        _means_body,
        out_shape=jax.ShapeDtypeStruct((2 * B, 16), f32),
    )(x3, bl3, spa_rW, spa_rb.reshape(1, E), spe_rW, spe_rb.reshape(1, E))

    mesh = plsc.VectorSubcoreMesh(core_axis_name="c", subcore_axis_name="s")
    sc_router = functools.partial(
        pl.kernel, mesh=mesh,
        out_type=(
            jax.ShapeDtypeStruct((2 * B, 16), jnp.int32),
            jax.ShapeDtypeStruct((2 * B, 16), f32),
        ),
        scratch_types=[
            pltpu.VMEM((16,), f32),
            pltpu.VMEM((16,), jnp.int32),
            pltpu.VMEM((16,), f32),
        ],
    )(_sc_router_body)
    idx_g, gate_g = sc_router(logits)

    grid_spec = pltpu.PrefetchScalarGridSpec(
        num_scalar_prefetch=2,
        grid=(B, J),
        in_specs=[
            pl.BlockSpec((1, CS, D_MODEL), lambda b, j, ig, gg: (b, 0, 0)),
            pl.BlockSpec((BF, D_MODEL), lambda b, j, ig, gg: (j, 0)),
            pl.BlockSpec((1, 1, BF), lambda b, j, ig, gg: (0, 0, j)),
            pl.BlockSpec((D_MODEL, BF), lambda b, j, ig, gg: (0, j)),
            pl.BlockSpec((1, BF, D_MODEL),
                         lambda b, j, ig, gg: (ig[b, 0], j, 0)),
            pl.BlockSpec((1, D_MODEL, BF),
                         lambda b, j, ig, gg: (ig[b, 0], 0, j)),
            pl.BlockSpec((1, BF, D_MODEL),
                         lambda b, j, ig, gg: (ig[B + b, 0], j, 0)),
            pl.BlockSpec((1, D_MODEL, BF),
                         lambda b, j, ig, gg: (ig[B + b, 0], 0, j)),
            pl.BlockSpec((1, 1, BF), lambda b, j, ig, gg: (ig[b, 0], 0, j)),
            pl.BlockSpec((1, 1, BF), lambda b, j, ig, gg: (ig[B + b, 0], 0, j)),
            pl.BlockSpec((1, 1, D_MODEL), lambda b, j, ig, gg: (0, 0, 0)),
            pl.BlockSpec((1, 1, D_MODEL),
                         lambda b, j, ig, gg: (ig[b, 0], 0, 0)),
            pl.BlockSpec((1, 1, D_MODEL),
                         lambda b, j, ig, gg: (ig[B + b, 0], 0, 0)),
        ],
        out_specs=pl.BlockSpec((1, CS, D_MODEL),
                               lambda b, j, ig, gg: (b, 0, 0)),
    )

    out = pl.pallas_call(
        _ffn_body,
        grid_spec=grid_spec,
        out_shape=jax.ShapeDtypeStruct((B, CS, D_MODEL), f32),
        compiler_params=pltpu.CompilerParams(
            dimension_semantics=("arbitrary", "arbitrary"),
            vmem_limit_bytes=100 * 1024 * 1024),
    )(idx_g, gate_g,
      x3, shared_W1, shared_b1.reshape(1, 1, D_FF), shared_W2,
      spa_W1, spa_W2, spe_W1, spe_W2,
      spa_b1.reshape(E, 1, D_FF), spe_b1.reshape(E, 1, D_FF),
      shared_b2.reshape(1, 1, D_MODEL),
      spa_b2.reshape(E, 1, D_MODEL), spe_b2.reshape(E, 1, D_MODEL))

    return out.reshape(B, C, S, D_MODEL)
